# baseline (speedup 1.0000x reference)
import functools
import math

import jax
import jax.numpy as jnp
from jax.experimental import pallas as pl
from jax.experimental.pallas import tpu as pltpu
from jax.experimental.pallas import tpu_sc as plsc

D_MODEL = 128
SCALE = math.sqrt(D_MODEL)
WINDOW = 64


def _scale_lut(lut):
    v, d = lut.shape
    blk = 2000

    def body(l_ref, o_ref):
        o_ref[...] = l_ref[...] * SCALE

    return pl.pallas_call(
        body,
        out_shape=jax.ShapeDtypeStruct((v, d), jnp.float32),
        grid=(v // blk,),
        in_specs=[pl.BlockSpec((blk, d), lambda i: (i, 0))],
        out_specs=pl.BlockSpec((blk, d), lambda i: (i, 0)),
    )(lut)


NC = 2
NS = 16
NW = NC * NS


RING = 8


def _sc_gather(scaled_lut, idx):
    from jax import lax

    b = idx.shape[0]
    b_per_w = b // NW
    n_chunks = b_per_w // WINDOW
    assert n_chunks % RING == 0 and n_chunks > RING
    mesh = plsc.VectorSubcoreMesh(core_axis_name="c", subcore_axis_name="s")

    scratch = (
        [pltpu.VMEM((WINDOW,), jnp.int32) for _ in range(RING)]
        + [pltpu.VMEM((WINDOW, D_MODEL), jnp.float32) for _ in range(RING)]
        + [pltpu.SemaphoreType.DMA for _ in range(2 * RING)]
    )

    @functools.partial(
        pl.kernel,
        out_type=jax.ShapeDtypeStruct((b, D_MODEL), jnp.float32),
        mesh=mesh,
        scratch_types=scratch,
    )
    def k(lut_hbm, i_hbm, o_hbm, *scr):
        idx_v = scr[:RING]
        buf = scr[RING : 2 * RING]
        gsem = scr[2 * RING : 3 * RING]
        osem = scr[3 * RING :]

        wid = lax.axis_index("c") * NS + lax.axis_index("s")
        base = wid * b_per_w

        def gather(j, bslot):
            return pltpu.make_async_copy(
                lut_hbm.at[idx_v[bslot]], buf[bslot], gsem[bslot]
            )

        def out_copy(j, bslot):
            return pltpu.make_async_copy(
                buf[bslot], o_hbm.at[pl.ds(base + j * WINDOW, WINDOW)], osem[bslot]
            )

        def load_idx(j, bslot):
            pltpu.sync_copy(i_hbm.at[pl.ds(base + j * WINDOW, WINDOW)], idx_v[bslot])

        for s in range(RING):
            load_idx(s, s)
            gather(s, s).start()

        @pl.loop(0, n_chunks - RING, step=RING)
        def _(g):
            for s in range(RING):
                gather(g + s, s).wait()
                out_copy(g + s, s).start()
            for s in range(RING):
                out_copy(g + s, s).wait()
                load_idx(g + RING + s, s)
                gather(g + RING + s, s).start()

        for s in range(RING):
            g = n_chunks - RING + s
            gather(g, s).wait()
            out_copy(g, s).start()
        for s in range(RING):
            out_copy(n_chunks - RING + s, s).wait()

    return k(scaled_lut, idx)


def kernel(x, lut):
    rows, cols = x.shape
    idx = x.reshape(-1).astype(jnp.int32)
    scaled = _scale_lut(lut)
    out = _sc_gather(scaled, idx)
    return out.reshape(rows, cols, D_MODEL)


# speedup vs baseline: 1.0862x; 1.0862x over previous
import functools
import math

import jax
import jax.numpy as jnp
from jax.experimental import pallas as pl
from jax.experimental.pallas import tpu as pltpu
from jax.experimental.pallas import tpu_sc as plsc

D_MODEL = 128
SCALE = math.sqrt(D_MODEL)
WINDOW = 128


def _scale_lut(lut):
    v, d = lut.shape
    blk = 2000

    def body(l_ref, o_ref):
        o_ref[...] = l_ref[...] * SCALE

    return pl.pallas_call(
        body,
        out_shape=jax.ShapeDtypeStruct((v, d), jnp.float32),
        grid=(v // blk,),
        in_specs=[pl.BlockSpec((blk, d), lambda i: (i, 0))],
        out_specs=pl.BlockSpec((blk, d), lambda i: (i, 0)),
    )(lut)


NC = 2
NS = 16
NW = NC * NS


RING = 5


def _sc_gather(scaled_lut, idx):
    from jax import lax

    b = idx.shape[0]
    b_per_w = b // NW
    n_chunks = b_per_w // WINDOW
    assert n_chunks % RING == 0 and n_chunks > RING
    mesh = plsc.VectorSubcoreMesh(core_axis_name="c", subcore_axis_name="s")

    scratch = (
        [pltpu.VMEM((b_per_w,), jnp.int32)]
        + [pltpu.VMEM((WINDOW, D_MODEL), jnp.float32) for _ in range(RING)]
        + [pltpu.SemaphoreType.DMA for _ in range(2 * RING)]
    )

    @functools.partial(
        pl.kernel,
        out_type=jax.ShapeDtypeStruct((b, D_MODEL), jnp.float32),
        mesh=mesh,
        scratch_types=scratch,
    )
    def k(lut_hbm, i_hbm, o_hbm, *scr):
        idx_all = scr[0]
        buf = scr[1 : 1 + RING]
        gsem = scr[1 + RING : 1 + 2 * RING]
        osem = scr[1 + 2 * RING :]

        wid = lax.axis_index("c") * NS + lax.axis_index("s")
        base = wid * b_per_w

        def gather(j, bslot):
            return pltpu.make_async_copy(
                lut_hbm.at[idx_all.at[pl.ds(j * WINDOW, WINDOW)]],
                buf[bslot],
                gsem[bslot],
            )

        def out_copy(j, bslot):
            return pltpu.make_async_copy(
                buf[bslot], o_hbm.at[pl.ds(base + j * WINDOW, WINDOW)], osem[bslot]
            )

        pltpu.sync_copy(i_hbm.at[pl.ds(base, b_per_w)], idx_all)

        for s in range(RING):
            gather(s, s).start()

        @pl.loop(0, n_chunks - RING, step=RING)
        def _(g):
            for s in range(RING):
                gather(g + s, s).wait()
                out_copy(g + s, s).start()
            for s in range(RING):
                out_copy(g + s, s).wait()
                gather(g + RING + s, s).start()

        for s in range(RING):
            g = n_chunks - RING + s
            gather(g, s).wait()
            out_copy(g, s).start()
        for s in range(RING):
            out_copy(n_chunks - RING + s, s).wait()

    return k(scaled_lut, idx)


def kernel(x, lut):
    rows, cols = x.shape
    idx = x.reshape(-1).astype(jnp.int32)
    scaled = _scale_lut(lut)
    out = _sc_gather(scaled, idx)
    return out.reshape(rows, cols, D_MODEL)


# speedup vs baseline: 1.0969x; 1.0099x over previous
import functools
import math

import jax
import jax.numpy as jnp
from jax.experimental import pallas as pl
from jax.experimental.pallas import tpu as pltpu
from jax.experimental.pallas import tpu_sc as plsc

D_MODEL = 128
SCALE = math.sqrt(D_MODEL)
WINDOW = 128


def _scale_lut(lut):
    v, d = lut.shape
    blk = 2000

    def body(l_ref, o_ref):
        o_ref[...] = l_ref[...] * SCALE

    return pl.pallas_call(
        body,
        out_shape=jax.ShapeDtypeStruct((v, d), jnp.float32),
        grid=(v // blk,),
        in_specs=[pl.BlockSpec((blk, d), lambda i: (i, 0))],
        out_specs=pl.BlockSpec((blk, d), lambda i: (i, 0)),
    )(lut)


NC = 2
NS = 16
NW = NC * NS


RING = 5
LAG = 3


def _sc_gather(scaled_lut, idx):
    from jax import lax

    b = idx.shape[0]
    b_per_w = b // NW
    n_chunks = b_per_w // WINDOW
    assert n_chunks % RING == 0 and n_chunks > RING
    mesh = plsc.VectorSubcoreMesh(core_axis_name="c", subcore_axis_name="s")

    scratch = (
        [pltpu.VMEM((b_per_w,), jnp.int32)]
        + [pltpu.VMEM((WINDOW, D_MODEL), jnp.float32) for _ in range(RING)]
        + [pltpu.SemaphoreType.DMA for _ in range(2 * RING)]
    )

    @functools.partial(
        pl.kernel,
        out_type=jax.ShapeDtypeStruct((b, D_MODEL), jnp.float32),
        mesh=mesh,
        scratch_types=scratch,
    )
    def k(lut_hbm, i_hbm, o_hbm, *scr):
        idx_all = scr[0]
        buf = scr[1 : 1 + RING]
        gsem = scr[1 + RING : 1 + 2 * RING]
        osem = scr[1 + 2 * RING :]

        wid = lax.axis_index("c") * NS + lax.axis_index("s")
        base = wid * b_per_w

        def gather(j, bslot):
            return pltpu.make_async_copy(
                lut_hbm.at[idx_all.at[pl.ds(j * WINDOW, WINDOW)]],
                buf[bslot],
                gsem[bslot],
            )

        def out_copy(j, bslot):
            return pltpu.make_async_copy(
                buf[bslot], o_hbm.at[pl.ds(base + j * WINDOW, WINDOW)], osem[bslot]
            )

        pltpu.sync_copy(i_hbm.at[pl.ds(base, b_per_w)], idx_all)

        for j in range(RING):
            gather(j, j).start()
            if j >= LAG:
                gather(j - LAG, j - LAG).wait()
                out_copy(j - LAG, j - LAG).start()

        @pl.loop(RING, n_chunks, step=RING)
        def _(g):
            for s in range(RING):
                j = g + s
                out_copy(j - RING, s).wait()
                gather(j, s).start()
                ls = (s - LAG) % RING
                gather(j - LAG, ls).wait()
                out_copy(j - LAG, ls).start()

        for j in range(n_chunks, n_chunks + LAG):
            ls = (j - LAG) % RING
            gather(j - LAG, ls).wait()
            out_copy(j - LAG, ls).start()
        for j in range(n_chunks - RING, n_chunks):
            out_copy(j, j % RING).wait()

    return k(scaled_lut, idx)


def kernel(x, lut):
    rows, cols = x.shape
    idx = x.reshape(-1).astype(jnp.int32)
    scaled = _scale_lut(lut)
    out = _sc_gather(scaled, idx)
    return out.reshape(rows, cols, D_MODEL)


# speedup vs baseline: 1.2652x; 1.1534x over previous
import functools
import math

import jax
import jax.numpy as jnp
from jax.experimental import pallas as pl
from jax.experimental.pallas import tpu as pltpu
from jax.experimental.pallas import tpu_sc as plsc

D_MODEL = 128
SCALE = math.sqrt(D_MODEL)
WINDOW = 128


NC = 2
NS = 16
NW = NC * NS


RING = 5
LAG = 3


def _sc_gather(scaled_lut, idx):
    from jax import lax

    b = idx.shape[0]
    b_per_w = b // NW
    n_chunks = b_per_w // WINDOW
    assert n_chunks % RING == 0 and n_chunks > RING
    mesh = plsc.VectorSubcoreMesh(core_axis_name="c", subcore_axis_name="s")

    scratch = (
        [pltpu.VMEM((b_per_w,), jnp.int32)]
        + [pltpu.VMEM((WINDOW, D_MODEL), jnp.float32) for _ in range(RING)]
        + [pltpu.SemaphoreType.DMA for _ in range(2 * RING)]
    )

    @functools.partial(
        pl.kernel,
        out_type=jax.ShapeDtypeStruct((b, D_MODEL), jnp.float32),
        mesh=mesh,
        scratch_types=scratch,
    )
    def k(lut_hbm, i_hbm, o_hbm, *scr):
        idx_all = scr[0]
        buf = scr[1 : 1 + RING]
        gsem = scr[1 + RING : 1 + 2 * RING]
        osem = scr[1 + 2 * RING :]

        wid = lax.axis_index("c") * NS + lax.axis_index("s")
        base = wid * b_per_w

        def gather(j, bslot):
            return pltpu.make_async_copy(
                lut_hbm.at[idx_all.at[pl.ds(j * WINDOW, WINDOW)]],
                buf[bslot],
                gsem[bslot],
            )

        def scale_buf(bslot):
            bref = buf[bslot]

            @pl.loop(0, WINDOW)
            def _(r):
                for c in range(0, D_MODEL, 16):
                    bref[r, pl.ds(c, 16)] = bref[r, pl.ds(c, 16)] * SCALE

        def out_copy(j, bslot):
            return pltpu.make_async_copy(
                buf[bslot], o_hbm.at[pl.ds(base + j * WINDOW, WINDOW)], osem[bslot]
            )

        pltpu.sync_copy(i_hbm.at[pl.ds(base, b_per_w)], idx_all)

        for j in range(RING):
            gather(j, j).start()
            if j >= LAG:
                gather(j - LAG, j - LAG).wait()
                scale_buf(j - LAG)
                out_copy(j - LAG, j - LAG).start()

        @pl.loop(RING, n_chunks, step=RING)
        def _(g):
            for s in range(RING):
                j = g + s
                out_copy(j - RING, s).wait()
                gather(j, s).start()
                ls = (s - LAG) % RING
                gather(j - LAG, ls).wait()
                scale_buf(ls)
                out_copy(j - LAG, ls).start()

        for j in range(n_chunks, n_chunks + LAG):
            ls = (j - LAG) % RING
            gather(j - LAG, ls).wait()
            scale_buf(ls)
            out_copy(j - LAG, ls).start()
        for j in range(n_chunks - RING, n_chunks):
            out_copy(j, j % RING).wait()

    return k(scaled_lut, idx)


def kernel(x, lut):
    rows, cols = x.shape
    idx = x.reshape(-1).astype(jnp.int32)
    out = _sc_gather(lut, idx)
    return out.reshape(rows, cols, D_MODEL)
